# baseline (speedup 1.0000x reference)
import jax
import jax.numpy as jnp
from jax.experimental import pallas as pl

N_VARS = 64
EMBED_DIM = 64


def _bcast_kernel(emb_ref, out_ref):
    out_ref[...] = jnp.broadcast_to(
        emb_ref[...][None, None, :, :], out_ref.shape
    )


def kernel(x, channel_emb):
    B, n_patches, _ = x.shape
    out = pl.pallas_call(
        _bcast_kernel,
        grid=(B,),
        in_specs=[pl.BlockSpec((N_VARS, EMBED_DIM), lambda i: (0, 0))],
        out_specs=pl.BlockSpec(
            (1, n_patches, N_VARS, EMBED_DIM), lambda i: (i, 0, 0, 0)
        ),
        out_shape=jax.ShapeDtypeStruct(
            (B, n_patches, N_VARS, EMBED_DIM), channel_emb.dtype
        ),
    )(channel_emb)
    return out
